# Initial kernel scaffold; baseline (speedup 1.0000x reference)
import jax
import jax.numpy as jnp
from jax.experimental import pallas as pl


def kernel(sents, lengths, forgetting_factor):
    raise NotImplementedError("write your pallas kernel here")


# speedup vs baseline: 8.5217x; 8.5217x over previous
import functools

import jax
import jax.numpy as jnp
from jax import lax
from jax.experimental import pallas as pl
from jax.experimental.pallas import tpu as pltpu
from jax.experimental.pallas import tpu_sc as plsc

VOCAB = 256
LANES = 16


def kernel(sents, lengths, forgetting_factor):
    B, S, W = sents.shape
    N = B * S
    NC, NS = 2, 16
    NW = NC * NS
    tok_per_w = N // NW
    G = tok_per_w // LANES

    sents_flat = sents.reshape(N // LANES, LANES, W).transpose(0, 2, 1).reshape(N * W)
    alpha_vec = jnp.broadcast_to(
        forgetting_factor.astype(jnp.float32), (LANES,))

    mesh = plsc.VectorSubcoreMesh(core_axis_name="c", subcore_axis_name="s")

    @functools.partial(
        pl.kernel,
        mesh=mesh,
        out_type=jax.ShapeDtypeStruct((N * VOCAB,), jnp.float32),
        compiler_params=pltpu.CompilerParams(needs_layout_passes=False),
        scratch_types=[
            pltpu.VMEM((tok_per_w * W,), jnp.int32),
            pltpu.VMEM((LANES,), jnp.float32),
            pltpu.VMEM((LANES * VOCAB,), jnp.float32),
        ],
    )
    def fofe(sents_hbm, alpha_hbm, out_hbm, chars_v, alpha_v, acc_v):
        wid = lax.axis_index("s") * NC + lax.axis_index("c")
        tok0 = wid * tok_per_w

        pltpu.sync_copy(sents_hbm.at[pl.ds(tok0 * W, tok_per_w * W)], chars_v)
        pltpu.sync_copy(alpha_hbm, alpha_v)

        alpha = alpha_v[...]
        lane = lax.iota(jnp.int32, 16)
        lane_vocab = lane * VOCAB
        zeros16 = jnp.zeros((LANES,), jnp.float32)
        ones16 = jnp.ones((LANES,), jnp.float32)

        def group_body(g, _):
            def zero_body(k, _):
                for j in range(8):
                    acc_v[pl.ds(k * 8 * LANES + j * LANES, LANES)] = zeros16
                return _
            lax.fori_loop(0, (LANES * VOCAB) // (8 * LANES), zero_body, None)

            base = g * (LANES * W)
            p = ones16
            for w in range(W - 1, -1, -1):
                c = chars_v[pl.ds(base + w * LANES, LANES)]
                m = c != 0
                plsc.addupdate_scatter(acc_v, [lane_vocab + c], p, mask=m)
                p = jnp.where(m, p * alpha, p)

            pltpu.sync_copy(
                acc_v,
                out_hbm.at[pl.ds((tok0 + g * LANES) * VOCAB, LANES * VOCAB)])
            return _

        lax.fori_loop(0, G, group_body, None)

    out = fofe(sents_flat, alpha_vec)
    return (out.reshape(B, S, VOCAB), lengths)


# speedup vs baseline: 9.6045x; 1.1271x over previous
import functools

import jax
import jax.numpy as jnp
from jax import lax
from jax.experimental import pallas as pl
from jax.experimental.pallas import tpu as pltpu
from jax.experimental.pallas import tpu_sc as plsc

VOCAB = 256
LANES = 16


def kernel(sents, lengths, forgetting_factor):
    B, S, W = sents.shape
    N = B * S
    NC, NS = 2, 16
    NW = NC * NS
    tok_per_w = N // NW
    G = tok_per_w // LANES

    sents_flat = sents.reshape(N * W)
    alpha_vec = jnp.broadcast_to(
        forgetting_factor.astype(jnp.float32), (LANES,))

    mesh = plsc.VectorSubcoreMesh(core_axis_name="c", subcore_axis_name="s")

    @functools.partial(
        pl.kernel,
        mesh=mesh,
        out_type=jax.ShapeDtypeStruct((N * VOCAB,), jnp.float32),
        compiler_params=pltpu.CompilerParams(needs_layout_passes=False),
        scratch_types=[
            pltpu.VMEM((tok_per_w * W,), jnp.int32),
            pltpu.VMEM((LANES,), jnp.float32),
            pltpu.VMEM((LANES * VOCAB,), jnp.float32),
        ],
    )
    def fofe(sents_hbm, alpha_hbm, out_hbm, chars_v, alpha_v, acc_v):
        wid = lax.axis_index("s") * NC + lax.axis_index("c")
        tok0 = wid * tok_per_w

        pltpu.sync_copy(sents_hbm.at[pl.ds(tok0 * W, tok_per_w * W)], chars_v)
        pltpu.sync_copy(alpha_hbm, alpha_v)

        alpha = alpha_v[...]
        lane = lax.iota(jnp.int32, 16)
        lane_w = lane * W
        lane_vocab = lane * VOCAB
        zeros16 = jnp.zeros((LANES,), jnp.float32)
        ones16 = jnp.ones((LANES,), jnp.float32)

        def group_body(g, _):
            def zero_body(k, _):
                for j in range(8):
                    acc_v[pl.ds(k * 8 * LANES + j * LANES, LANES)] = zeros16
                return _
            lax.fori_loop(0, (LANES * VOCAB) // (8 * LANES), zero_body, None)

            base = g * (LANES * W)
            p = ones16
            for w in range(W - 1, -1, -1):
                c = plsc.load_gather(chars_v, [lane_w + (base + w)])
                m = c != 0
                plsc.addupdate_scatter(acc_v, [lane_vocab + c], p, mask=m)
                p = jnp.where(m, p * alpha, p)

            pltpu.sync_copy(
                acc_v,
                out_hbm.at[pl.ds((tok0 + g * LANES) * VOCAB, LANES * VOCAB)])
            return _

        lax.fori_loop(0, G, group_body, None)

    out = fofe(sents_flat, alpha_vec)
    return (out.reshape(B, S, VOCAB), lengths)


# speedup vs baseline: 10.3256x; 1.0751x over previous
import functools

import jax
import jax.numpy as jnp
from jax import lax
from jax.experimental import pallas as pl
from jax.experimental.pallas import tpu as pltpu
from jax.experimental.pallas import tpu_sc as plsc

VOCAB = 256
LANES = 16


def kernel(sents, lengths, forgetting_factor):
    B, S, W = sents.shape
    N = B * S
    NC, NS = 2, 16
    NW = NC * NS
    tok_per_w = N // NW
    G = tok_per_w // LANES
    BUF = LANES * VOCAB

    sents_flat = sents.reshape(N * W)
    alpha_vec = jnp.broadcast_to(
        forgetting_factor.astype(jnp.float32), (LANES,))

    mesh = plsc.VectorSubcoreMesh(core_axis_name="c", subcore_axis_name="s")

    @functools.partial(
        pl.kernel,
        mesh=mesh,
        out_type=jax.ShapeDtypeStruct((N * VOCAB,), jnp.float32),
        compiler_params=pltpu.CompilerParams(needs_layout_passes=False),
        scratch_types=[
            pltpu.VMEM((tok_per_w * W,), jnp.int32),
            pltpu.VMEM((LANES,), jnp.float32),
            pltpu.VMEM((BUF,), jnp.float32),
            pltpu.VMEM((BUF,), jnp.float32),
            pltpu.SemaphoreType.DMA,
            pltpu.SemaphoreType.DMA,
        ],
    )
    def fofe(sents_hbm, alpha_hbm, out_hbm, chars_v, alpha_v, acc_a, acc_b,
             sem_a, sem_b):
        wid = lax.axis_index("s") * NC + lax.axis_index("c")
        tok0 = wid * tok_per_w

        pltpu.sync_copy(sents_hbm.at[pl.ds(tok0 * W, tok_per_w * W)], chars_v)
        pltpu.sync_copy(alpha_hbm, alpha_v)

        alpha = alpha_v[...]
        lane = lax.iota(jnp.int32, 16)
        lane_w = lane * W
        lane_vocab = lane * VOCAB
        zeros16 = jnp.zeros((LANES,), jnp.float32)
        ones16 = jnp.ones((LANES,), jnp.float32)

        bufs = (acc_a, acc_b)
        sems = (sem_a, sem_b)

        def zero_body(k, _):
            for j in range(8):
                acc_a[pl.ds(k * 8 * LANES + j * LANES, LANES)] = zeros16
                acc_b[pl.ds(k * 8 * LANES + j * LANES, LANES)] = zeros16
            return _
        lax.fori_loop(0, BUF // (8 * LANES), zero_body, None)

        dma = [None, None]
        prev_chars = [None, None]
        for g in range(G):
            b = g & 1
            acc = bufs[b]
            if dma[b] is not None:
                dma[b].wait()
                for c in prev_chars[b]:
                    plsc.store_scatter(acc, [lane_vocab + c], zeros16,
                                       mask=c != 0)

            base = g * (LANES * W)
            cs = [plsc.load_gather(chars_v, [lane_w + (base + w)])
                  for w in range(W - 1, -1, -1)]
            p = ones16
            for c in cs:
                m = c != 0
                plsc.addupdate_scatter(acc, [lane_vocab + c], p, mask=m)
                p = jnp.where(m, p * alpha, p)
            prev_chars[b] = cs

            dma[b] = pltpu.async_copy(
                acc, out_hbm.at[pl.ds((tok0 + g * LANES) * VOCAB, BUF)],
                sems[b])

        dma[0].wait()
        dma[1].wait()

    out = fofe(sents_flat, alpha_vec)
    return (out.reshape(B, S, VOCAB), lengths)


# speedup vs baseline: 14.0720x; 1.3628x over previous
import functools

import jax
import jax.numpy as jnp
from jax import lax
from jax.experimental import pallas as pl
from jax.experimental.pallas import tpu as pltpu
from jax.experimental.pallas import tpu_sc as plsc

VOCAB = 256
LANES = 16


def kernel(sents, lengths, forgetting_factor):
    B, S, W = sents.shape
    N = B * S
    NC, NS = 2, 16
    NW = NC * NS
    tok_per_w = N // NW
    G = tok_per_w // LANES
    s_per_w = S // (NW // B) if NW >= B else S * (B // NW)

    alpha_vec = jnp.broadcast_to(
        forgetting_factor.astype(jnp.float32), (LANES,))

    mesh = plsc.VectorSubcoreMesh(core_axis_name="c", subcore_axis_name="s")

    @functools.partial(
        pl.kernel,
        mesh=mesh,
        out_type=jax.ShapeDtypeStruct((B, S, VOCAB), jnp.float32),
        compiler_params=pltpu.CompilerParams(needs_layout_passes=False),
        scratch_types=[
            pltpu.VMEM((tok_per_w, W), jnp.int32),
            pltpu.VMEM((LANES,), jnp.float32),
            pltpu.VMEM((LANES, VOCAB), jnp.float32),
            pltpu.VMEM((LANES, VOCAB), jnp.float32),
            pltpu.SemaphoreType.DMA,
            pltpu.SemaphoreType.DMA,
        ],
    )
    def fofe(sents_hbm, alpha_hbm, out_hbm, chars_v, alpha_v, acc_a, acc_b,
             sem_a, sem_b):
        wid = lax.axis_index("s") * NC + lax.axis_index("c")
        batch = wid // (NW // B)
        s_base = (wid % (NW // B)) * tok_per_w

        pltpu.sync_copy(sents_hbm.at[batch, pl.ds(s_base, tok_per_w)], chars_v)
        pltpu.sync_copy(alpha_hbm, alpha_v)

        alpha = alpha_v[...]
        lane = lax.iota(jnp.int32, 16)
        zeros16 = jnp.zeros((LANES,), jnp.float32)
        ones16 = jnp.ones((LANES,), jnp.float32)

        bufs = (acc_a, acc_b)
        sems = (sem_a, sem_b)

        def zero_body(k, _):
            for r in range(LANES):
                acc_a[r, pl.ds(k * LANES, LANES)] = zeros16
                acc_b[r, pl.ds(k * LANES, LANES)] = zeros16
            return _
        lax.fori_loop(0, VOCAB // LANES, zero_body, None)

        dma = [None, None]
        prev_chars = [None, None]
        for g in range(G):
            b = g & 1
            acc = bufs[b]
            if dma[b] is not None:
                dma[b].wait()
                for c in prev_chars[b]:
                    plsc.store_scatter(acc, [lane, c], zeros16, mask=c != 0)

            tok = lane + g * LANES
            cs = [plsc.load_gather(chars_v, [tok, jnp.full((LANES,), w,
                                                           jnp.int32)])
                  for w in range(W - 1, -1, -1)]
            p = ones16
            for c in cs:
                m = c != 0
                plsc.addupdate_scatter(acc, [lane, c], p, mask=m)
                p = jnp.where(m, p * alpha, p)
            prev_chars[b] = cs

            dma[b] = pltpu.async_copy(
                acc, out_hbm.at[batch, pl.ds(s_base + g * LANES, LANES)],
                sems[b])

        dma[0].wait()
        dma[1].wait()

    out = fofe(sents, alpha_vec)
    return (out, lengths)


# speedup vs baseline: 14.0971x; 1.0018x over previous
import functools

import jax
import jax.numpy as jnp
from jax import lax
from jax.experimental import pallas as pl
from jax.experimental.pallas import tpu as pltpu
from jax.experimental.pallas import tpu_sc as plsc

VOCAB = 256
LANES = 16


def kernel(sents, lengths, forgetting_factor):
    B, S, W = sents.shape
    N = B * S
    NC, NS = 2, 16
    NW = NC * NS
    tok_per_w = N // NW
    G = tok_per_w // LANES
    s_per_w = S // (NW // B) if NW >= B else S * (B // NW)

    alpha_vec = jnp.broadcast_to(
        forgetting_factor.astype(jnp.float32), (LANES,))

    mesh = plsc.VectorSubcoreMesh(core_axis_name="c", subcore_axis_name="s")

    @functools.partial(
        pl.kernel,
        mesh=mesh,
        out_type=jax.ShapeDtypeStruct((B, S, VOCAB), jnp.float32),
        compiler_params=pltpu.CompilerParams(
            needs_layout_passes=False, use_tc_tiling_on_sc=True),
        scratch_types=[
            pltpu.VMEM((tok_per_w, W), jnp.int32),
            pltpu.VMEM((LANES,), jnp.float32),
            pltpu.VMEM((LANES, VOCAB), jnp.float32),
            pltpu.VMEM((LANES, VOCAB), jnp.float32),
            pltpu.SemaphoreType.DMA,
            pltpu.SemaphoreType.DMA,
        ],
    )
    def fofe(sents_hbm, alpha_hbm, out_hbm, chars_v, alpha_v, acc_a, acc_b,
             sem_a, sem_b):
        wid = lax.axis_index("s") * NC + lax.axis_index("c")
        batch = wid // (NW // B)
        s_base = (wid % (NW // B)) * tok_per_w

        pltpu.sync_copy(sents_hbm.at[batch, pl.ds(s_base, tok_per_w)], chars_v)
        pltpu.sync_copy(alpha_hbm, alpha_v)

        alpha = alpha_v[...]
        lane = lax.iota(jnp.int32, 16)
        zeros16 = jnp.zeros((LANES,), jnp.float32)
        ones16 = jnp.ones((LANES,), jnp.float32)

        bufs = (acc_a, acc_b)
        sems = (sem_a, sem_b)

        def zero_body(k, _):
            for r in range(LANES):
                acc_a[r, pl.ds(k * LANES, LANES)] = zeros16
                acc_b[r, pl.ds(k * LANES, LANES)] = zeros16
            return _
        lax.fori_loop(0, VOCAB // LANES, zero_body, None)

        dma = [None, None]
        prev_chars = [None, None]
        for g in range(G):
            b = g & 1
            acc = bufs[b]
            if dma[b] is not None:
                dma[b].wait()
                for c in prev_chars[b]:
                    plsc.store_scatter(acc, [lane, c], zeros16, mask=c != 0)

            tok = lane + g * LANES
            cs = [plsc.load_gather(chars_v, [tok, jnp.full((LANES,), w,
                                                           jnp.int32)])
                  for w in range(W - 1, -1, -1)]
            p = ones16
            for c in cs:
                m = c != 0
                plsc.addupdate_scatter(acc, [lane, c], p, mask=m)
                p = jnp.where(m, p * alpha, p)
            prev_chars[b] = cs

            dma[b] = pltpu.async_copy(
                acc, out_hbm.at[batch, pl.ds(s_base + g * LANES, LANES)],
                sems[b])

        dma[0].wait()
        dma[1].wait()

    out = fofe(sents, alpha_vec)
    return (out, lengths)
